# Initial kernel scaffold; baseline (speedup 1.0000x reference)
import jax
import jax.numpy as jnp
from jax.experimental import pallas as pl


def kernel(image, ndvi, slic, Wc, b):
    raise NotImplementedError("write your pallas kernel here")


# speedup vs baseline: 93.9203x; 93.9203x over previous
import functools

import jax
import jax.numpy as jnp
from jax import lax
from jax.experimental import pallas as pl
from jax.experimental.pallas import tpu as pltpu
from jax.experimental.pallas import tpu_sc as plsc

B = 4
C = 3
H = 512
W = 512
HW = H * W
NPIX = B * HW
NSEG = 1024
NCLS = 3
NDVI_THRESH = 0.5

NWPB = 8
PPW = HW // NWPB
CH = 4096
NCHUNK = PPW // CH
VPC = CH // 16
LANES = 16


def _sc_body(img_hbm, ndvi_hbm, slic_hbm, wc_hbm, b_hbm, out_hbm,
             acc_r, acc_g, acc_b, acc_c,
             slic_buf, ndvi_buf, ch_r, ch_g, ch_b,
             partial, buf8, tbl_slice, tbl_local, out_buf,
             wc_v, bias_v, shared_acc, shared_tbl):
  core = lax.axis_index("c")
  sub = lax.axis_index("s")
  lb = sub // NWPB
  part = sub % NWPB
  batch = core * 2 + lb
  pix_base = batch * HW + part * PPW

  lane = lax.iota(jnp.int32, LANES)
  lane_base = lane * NSEG
  ones = jnp.full((LANES,), 1.0, dtype=jnp.float32)
  zf = jnp.zeros((LANES,), dtype=jnp.float32)

  def zero_body(i, _):
    for j in range(4):
      o = (i * 4 + j) * LANES
      acc_r[pl.ds(o, LANES)] = zf
      acc_g[pl.ds(o, LANES)] = zf
      acc_b[pl.ds(o, LANES)] = zf
      acc_c[pl.ds(o, LANES)] = zf
    return 0
  lax.fori_loop(0, (LANES * NSEG) // (4 * LANES), zero_body, 0)

  def p1_chunk(i, _):
    off = pix_base + i * CH
    pltpu.sync_copy(slic_hbm.at[pl.ds(off, CH)], slic_buf)
    pltpu.sync_copy(ndvi_hbm.at[pl.ds(off, CH)], ndvi_buf)
    coff = (batch * C) * HW + part * PPW + i * CH
    pltpu.sync_copy(img_hbm.at[pl.ds(coff, CH)], ch_r)
    pltpu.sync_copy(img_hbm.at[pl.ds(coff + HW, CH)], ch_g)
    pltpu.sync_copy(img_hbm.at[pl.ds(coff + 2 * HW, CH)], ch_b)

    def p1_vec(v, _):
      o = v * LANES
      sl = slic_buf[pl.ds(o, LANES)]
      nd = ndvi_buf[pl.ds(o, LANES)]
      m = nd > NDVI_THRESH
      seg = jnp.where(m, sl, 0)
      idx = seg + lane_base
      plsc.addupdate_scatter(acc_r, [idx], ch_r[pl.ds(o, LANES)])
      plsc.addupdate_scatter(acc_g, [idx], ch_g[pl.ds(o, LANES)])
      plsc.addupdate_scatter(acc_b, [idx], ch_b[pl.ds(o, LANES)])
      plsc.addupdate_scatter(acc_c, [idx], ones)
      return 0
    lax.fori_loop(0, VPC, p1_vec, 0)
    return 0
  lax.fori_loop(0, NCHUNK, p1_chunk, 0)

  def red_body(t, _):
    o = t * LANES
    for a, ref in enumerate((acc_r, acc_g, acc_b, acc_c)):
      tot = ref[pl.ds(o, LANES)]
      for l in range(1, LANES):
        tot = tot + ref[pl.ds(l * NSEG + o, LANES)]
      partial[a, pl.ds(o, LANES)] = tot
    return 0
  lax.fori_loop(0, NSEG // LANES, red_body, 0)

  pltpu.sync_copy(partial, shared_acc.at[sub])
  plsc.subcore_barrier()

  gbase = lb * NWPB
  soff = part * (NSEG // NWPB)
  pltpu.sync_copy(
      shared_acc.at[pl.ds(gbase, NWPB), :, pl.ds(soff, NSEG // NWPB)], buf8)

  pltpu.sync_copy(wc_hbm, wc_v.at[pl.ds(0, 9)])
  pltpu.sync_copy(b_hbm, bias_v.at[pl.ds(0, 3)])
  wc_vec = wc_v[pl.ds(0, LANES)]
  bias_vec = bias_v[pl.ds(0, LANES)]
  wc = [wc_vec[k] for k in range(9)]
  bs = [bias_vec[k] for k in range(3)]

  def p2_vec(t, _):
    o = t * LANES
    sums = []
    for a in range(4):
      tot = buf8[0, a, pl.ds(o, LANES)]
      for k in range(1, NWPB):
        tot = tot + buf8[k, a, pl.ds(o, LANES)]
      sums.append(tot)
    inv = 1.0 / jnp.maximum(sums[3], 1.0)
    mr = sums[0] * inv
    mg = sums[1] * inv
    mb = sums[2] * inv
    l0 = mr * wc[0] + mg * wc[3] + mb * wc[6] + bs[0]
    l1 = mr * wc[1] + mg * wc[4] + mb * wc[7] + bs[1]
    l2 = mr * wc[2] + mg * wc[5] + mb * wc[8] + bs[2]
    mx = jnp.maximum(l0, jnp.maximum(l1, l2))
    e0 = jnp.exp(l0 - mx)
    e1 = jnp.exp(l1 - mx)
    e2 = jnp.exp(l2 - mx)
    tbl_slice[pl.ds(o, LANES)] = (e1 + 2.0 * e2) / (e0 + e1 + e2) + 1.0
    return 0
  lax.fori_loop(0, (NSEG // NWPB) // LANES, p2_vec, 0)

  pltpu.sync_copy(tbl_slice, shared_tbl.at[lb, pl.ds(soff, NSEG // NWPB)])
  plsc.subcore_barrier()
  pltpu.sync_copy(shared_tbl.at[lb], tbl_local)

  def p3_chunk(i, _):
    off = pix_base + i * CH
    pltpu.sync_copy(slic_hbm.at[pl.ds(off, CH)], slic_buf)
    pltpu.sync_copy(ndvi_hbm.at[pl.ds(off, CH)], ndvi_buf)

    def p3_vec(v, _):
      o = v * LANES
      sl = slic_buf[pl.ds(o, LANES)]
      nd = ndvi_buf[pl.ds(o, LANES)]
      m = nd > NDVI_THRESH
      seg = jnp.where(m, sl, 0)
      lbl = plsc.load_gather(tbl_local, [seg])
      res = jnp.where(seg > 0, lbl, jnp.where(m, ones, zf))
      out_buf[pl.ds(o, LANES)] = res
      return 0
    lax.fori_loop(0, VPC, p3_vec, 0)
    pltpu.sync_copy(out_buf, out_hbm.at[pl.ds(off, CH)])
    return 0
  lax.fori_loop(0, NCHUNK, p3_chunk, 0)


@jax.jit
def _run(img_flat, ndvi_flat, slic_flat, wc_flat, bias):
  mesh = plsc.VectorSubcoreMesh(core_axis_name="c", subcore_axis_name="s")
  f = pl.kernel(
      _sc_body,
      out_type=jax.ShapeDtypeStruct((NPIX,), jnp.float32),
      mesh=mesh,
      compiler_params=pltpu.CompilerParams(needs_layout_passes=False),
      scratch_types=[
          pltpu.VMEM((LANES * NSEG,), jnp.float32),
          pltpu.VMEM((LANES * NSEG,), jnp.float32),
          pltpu.VMEM((LANES * NSEG,), jnp.float32),
          pltpu.VMEM((LANES * NSEG,), jnp.float32),
          pltpu.VMEM((CH,), jnp.int32),
          pltpu.VMEM((CH,), jnp.float32),
          pltpu.VMEM((CH,), jnp.float32),
          pltpu.VMEM((CH,), jnp.float32),
          pltpu.VMEM((CH,), jnp.float32),
          pltpu.VMEM((4, NSEG), jnp.float32),
          pltpu.VMEM((NWPB, 4, NSEG // NWPB), jnp.float32),
          pltpu.VMEM((NSEG // NWPB,), jnp.float32),
          pltpu.VMEM((NSEG,), jnp.float32),
          pltpu.VMEM((CH,), jnp.float32),
          pltpu.VMEM((LANES,), jnp.float32),
          pltpu.VMEM((LANES,), jnp.float32),
          pltpu.VMEM_SHARED((16, 4, NSEG), jnp.float32),
          pltpu.VMEM_SHARED((2, NSEG), jnp.float32),
      ],
  )
  return f(img_flat, ndvi_flat, slic_flat, wc_flat, bias)


def kernel(image, ndvi, slic, Wc, b):
  out = _run(image.reshape(-1), ndvi.reshape(-1), slic.reshape(-1),
             Wc.reshape(-1), b)
  return out.reshape(B, 1, H, W)


# speedup vs baseline: 108.9851x; 1.1604x over previous
import jax
import jax.numpy as jnp
from jax import lax
from jax.experimental import pallas as pl
from jax.experimental.pallas import tpu as pltpu
from jax.experimental.pallas import tpu_sc as plsc

B = 4
C = 3
H = 512
W = 512
HW = H * W
NPIX = B * HW
NSEG = 1024
NCLS = 3
NDVI_THRESH = 0.5

NWPB = 8
PPW = HW // NWPB
CH = 4096
NCHUNK = PPW // CH
VPC = CH // 16
LANES = 16
UNROLL = 4


def _sc_body(img_hbm, ndvi_hbm, slic_hbm, wc_hbm, b_hbm, out_hbm,
             acc_r, acc_g, acc_b, acc_c,
             slic_buf, ndvi_buf, ch_r, ch_g, ch_b,
             partial, buf8, tbl_slice, tbl_local, out_buf,
             wc_v, bias_v, shared_acc, shared_tbl,
             sem_a, sem_b, sem_o):
  core = lax.axis_index("c")
  sub = lax.axis_index("s")
  lb = sub // NWPB
  part = sub % NWPB
  batch = core * 2 + lb
  pix_base = batch * HW + part * PPW
  img_base = (batch * C) * HW + part * PPW

  lane = lax.iota(jnp.int32, LANES)
  lane_base = lane * NSEG
  ones = jnp.full((LANES,), 1.0, dtype=jnp.float32)
  zf = jnp.zeros((LANES,), dtype=jnp.float32)
  sems = (sem_a, sem_b)

  def fire_p1(i):
    par = i % 2
    off = pix_base + i * CH
    coff = img_base + i * CH
    sem = sems[par]
    return [
        pltpu.async_copy(slic_hbm.at[pl.ds(off, CH)], slic_buf.at[par], sem),
        pltpu.async_copy(ndvi_hbm.at[pl.ds(off, CH)], ndvi_buf.at[par], sem),
        pltpu.async_copy(img_hbm.at[pl.ds(coff, CH)], ch_r.at[par], sem),
        pltpu.async_copy(img_hbm.at[pl.ds(coff + HW, CH)], ch_g.at[par], sem),
        pltpu.async_copy(img_hbm.at[pl.ds(coff + 2 * HW, CH)], ch_b.at[par],
                         sem),
    ]

  descs = [fire_p1(0), None]

  def zero_body(i, _):
    for j in range(4):
      o = (i * 4 + j) * LANES
      acc_r[pl.ds(o, LANES)] = zf
      acc_g[pl.ds(o, LANES)] = zf
      acc_b[pl.ds(o, LANES)] = zf
      acc_c[pl.ds(o, LANES)] = zf
    return 0
  lax.fori_loop(0, (LANES * NSEG) // (4 * LANES), zero_body, 0)

  for i in range(NCHUNK):
    par = i % 2
    if i + 1 < NCHUNK:
      descs[1 - par] = fire_p1(i + 1)
    for d in descs[par]:
      d.wait()

    def p1_vec(v, _, par=par):
      for u in range(UNROLL):
        o = (v * UNROLL + u) * LANES
        sl = slic_buf[par, pl.ds(o, LANES)]
        nd = ndvi_buf[par, pl.ds(o, LANES)]
        m = nd > NDVI_THRESH
        seg = jnp.where(m, sl, 0)
        idx = seg + lane_base
        plsc.addupdate_scatter(acc_r, [idx], ch_r[par, pl.ds(o, LANES)])
        plsc.addupdate_scatter(acc_g, [idx], ch_g[par, pl.ds(o, LANES)])
        plsc.addupdate_scatter(acc_b, [idx], ch_b[par, pl.ds(o, LANES)])
        plsc.addupdate_scatter(acc_c, [idx], ones)
      return 0
    lax.fori_loop(0, VPC // UNROLL, p1_vec, 0)

  def fire_p3(i):
    par = i % 2
    off = pix_base + i * CH
    sem = sems[par]
    return [
        pltpu.async_copy(slic_hbm.at[pl.ds(off, CH)], slic_buf.at[par], sem),
        pltpu.async_copy(ndvi_hbm.at[pl.ds(off, CH)], ndvi_buf.at[par], sem),
    ]
  descs3 = [fire_p3(0), None]

  def red_body(t, _):
    o = t * LANES
    for a, ref in enumerate((acc_r, acc_g, acc_b, acc_c)):
      tot = ref[pl.ds(o, LANES)]
      for l in range(1, LANES):
        tot = tot + ref[pl.ds(l * NSEG + o, LANES)]
      partial[a, pl.ds(o, LANES)] = tot
    return 0
  lax.fori_loop(0, NSEG // LANES, red_body, 0)

  pltpu.sync_copy(partial, shared_acc.at[sub])
  plsc.subcore_barrier()

  gbase = lb * NWPB
  soff = part * (NSEG // NWPB)
  pltpu.sync_copy(
      shared_acc.at[pl.ds(gbase, NWPB), :, pl.ds(soff, NSEG // NWPB)], buf8)

  pltpu.sync_copy(wc_hbm, wc_v.at[pl.ds(0, 9)])
  pltpu.sync_copy(b_hbm, bias_v.at[pl.ds(0, 3)])
  wc_vec = wc_v[pl.ds(0, LANES)]
  bias_vec = bias_v[pl.ds(0, LANES)]
  wc = [wc_vec[k] for k in range(9)]
  bs = [bias_vec[k] for k in range(3)]

  def p2_vec(t, _):
    o = t * LANES
    sums = []
    for a in range(4):
      tot = buf8[0, a, pl.ds(o, LANES)]
      for k in range(1, NWPB):
        tot = tot + buf8[k, a, pl.ds(o, LANES)]
      sums.append(tot)
    inv = 1.0 / jnp.maximum(sums[3], 1.0)
    mr = sums[0] * inv
    mg = sums[1] * inv
    mb = sums[2] * inv
    l0 = mr * wc[0] + mg * wc[3] + mb * wc[6] + bs[0]
    l1 = mr * wc[1] + mg * wc[4] + mb * wc[7] + bs[1]
    l2 = mr * wc[2] + mg * wc[5] + mb * wc[8] + bs[2]
    mx = jnp.maximum(l0, jnp.maximum(l1, l2))
    e0 = jnp.exp(l0 - mx)
    e1 = jnp.exp(l1 - mx)
    e2 = jnp.exp(l2 - mx)
    tbl_slice[pl.ds(o, LANES)] = (e1 + 2.0 * e2) / (e0 + e1 + e2) + 1.0
    return 0
  lax.fori_loop(0, (NSEG // NWPB) // LANES, p2_vec, 0)

  pltpu.sync_copy(tbl_slice, shared_tbl.at[lb, pl.ds(soff, NSEG // NWPB)])
  plsc.subcore_barrier()
  pltpu.sync_copy(shared_tbl.at[lb], tbl_local)

  out_descs = [None, None]
  for i in range(NCHUNK):
    par = i % 2
    if i + 1 < NCHUNK:
      descs3[1 - par] = fire_p3(i + 1)
    for d in descs3[par]:
      d.wait()
    if out_descs[par] is not None:
      out_descs[par].wait()

    def p3_vec(v, _, par=par):
      for u in range(UNROLL):
        o = (v * UNROLL + u) * LANES
        sl = slic_buf[par, pl.ds(o, LANES)]
        nd = ndvi_buf[par, pl.ds(o, LANES)]
        m = nd > NDVI_THRESH
        seg = jnp.where(m, sl, 0)
        lbl = plsc.load_gather(tbl_local, [seg])
        res = jnp.where(seg > 0, lbl, jnp.where(m, ones, zf))
        out_buf[par, pl.ds(o, LANES)] = res
      return 0
    lax.fori_loop(0, VPC // UNROLL, p3_vec, 0)

    off = pix_base + i * CH
    out_descs[par] = pltpu.async_copy(
        out_buf.at[par], out_hbm.at[pl.ds(off, CH)], sem_o)
  out_descs[0].wait()
  out_descs[1].wait()


@jax.jit
def _run(img_flat, ndvi_flat, slic_flat, wc_flat, bias):
  mesh = plsc.VectorSubcoreMesh(core_axis_name="c", subcore_axis_name="s")
  f = pl.kernel(
      _sc_body,
      out_type=jax.ShapeDtypeStruct((NPIX,), jnp.float32),
      mesh=mesh,
      compiler_params=pltpu.CompilerParams(needs_layout_passes=False),
      scratch_types=[
          pltpu.VMEM((LANES * NSEG,), jnp.float32),
          pltpu.VMEM((LANES * NSEG,), jnp.float32),
          pltpu.VMEM((LANES * NSEG,), jnp.float32),
          pltpu.VMEM((LANES * NSEG,), jnp.float32),
          pltpu.VMEM((2, CH), jnp.int32),
          pltpu.VMEM((2, CH), jnp.float32),
          pltpu.VMEM((2, CH), jnp.float32),
          pltpu.VMEM((2, CH), jnp.float32),
          pltpu.VMEM((2, CH), jnp.float32),
          pltpu.VMEM((4, NSEG), jnp.float32),
          pltpu.VMEM((NWPB, 4, NSEG // NWPB), jnp.float32),
          pltpu.VMEM((NSEG // NWPB,), jnp.float32),
          pltpu.VMEM((NSEG,), jnp.float32),
          pltpu.VMEM((2, CH), jnp.float32),
          pltpu.VMEM((LANES,), jnp.float32),
          pltpu.VMEM((LANES,), jnp.float32),
          pltpu.VMEM_SHARED((16, 4, NSEG), jnp.float32),
          pltpu.VMEM_SHARED((2, NSEG), jnp.float32),
          pltpu.SemaphoreType.DMA,
          pltpu.SemaphoreType.DMA,
          pltpu.SemaphoreType.DMA,
      ],
  )
  return f(img_flat, ndvi_flat, slic_flat, wc_flat, bias)


def kernel(image, ndvi, slic, Wc, b):
  out = _run(image.reshape(-1), ndvi.reshape(-1), slic.reshape(-1),
             Wc.reshape(-1), b)
  return out.reshape(B, 1, H, W)


# speedup vs baseline: 140.6205x; 1.2903x over previous
import jax
import jax.numpy as jnp
from jax import lax
from jax.experimental import pallas as pl
from jax.experimental.pallas import tpu as pltpu
from jax.experimental.pallas import tpu_sc as plsc

B = 4
C = 3
H = 512
W = 512
HW = H * W
NPIX = B * HW
NSEG = 1024
NCLS = 3
NDVI_THRESH = 0.5

NWPB = 8
PPW = HW // NWPB
CH = 4096
NCHUNK = PPW // CH
VPC = CH // 16
LANES = 16
UNROLL = 4


def _sc_body(img_hbm, ndvi_hbm, slic_hbm, wc_hbm, b_hbm, out_hbm,
             acc_r, acc_g, acc_b, acc_c,
             slic_buf, ndvi_buf, ch_r, ch_g, ch_b,
             partial, buf8, tbl_slice, tbl_local, out_buf,
             wc_v, bias_v, shared_acc, shared_tbl,
             sem_a, sem_b, sem_o):
  core = lax.axis_index("c")
  sub = lax.axis_index("s")
  lb = sub // NWPB
  part = sub % NWPB
  batch = core * 2 + lb
  pix_base = batch * HW + part * PPW
  img_base = (batch * C) * HW + part * PPW

  lane = lax.iota(jnp.int32, LANES)
  lane_base = lane * NSEG
  ones = jnp.full((LANES,), 1.0, dtype=jnp.float32)
  zf = jnp.zeros((LANES,), dtype=jnp.float32)
  sems = (sem_a, sem_b)

  def fire_p1(i):
    par = i % 2
    off = pix_base + i * CH
    coff = img_base + i * CH
    sem = sems[par]
    return [
        pltpu.async_copy(slic_hbm.at[pl.ds(off, CH)], slic_buf.at[par], sem),
        pltpu.async_copy(ndvi_hbm.at[pl.ds(off, CH)], ndvi_buf.at[par], sem),
        pltpu.async_copy(img_hbm.at[pl.ds(coff, CH)], ch_r.at[par], sem),
        pltpu.async_copy(img_hbm.at[pl.ds(coff + HW, CH)], ch_g.at[par], sem),
        pltpu.async_copy(img_hbm.at[pl.ds(coff + 2 * HW, CH)], ch_b.at[par],
                         sem),
    ]

  descs = [fire_p1(0), None]

  @plsc.parallel_loop(0, LANES * NSEG, step=4 * LANES, unroll=4)
  def _(i):
    for j in range(4):
      o = i + j * LANES
      acc_r[pl.ds(o, LANES)] = zf
      acc_g[pl.ds(o, LANES)] = zf
      acc_b[pl.ds(o, LANES)] = zf
      acc_c[pl.ds(o, LANES)] = zf

  for i in range(NCHUNK):
    par = i % 2
    if i + 1 < NCHUNK:
      descs[1 - par] = fire_p1(i + 1)
    for d in descs[par]:
      d.wait()

    @plsc.parallel_loop(0, CH, step=LANES, unroll=UNROLL)
    def _(o, par=par):
      sl = slic_buf[par, pl.ds(o, LANES)]
      nd = ndvi_buf[par, pl.ds(o, LANES)]
      m = nd > NDVI_THRESH
      seg = jnp.where(m, sl, 0)
      idx = seg + lane_base
      plsc.addupdate_scatter(acc_r, [idx], ch_r[par, pl.ds(o, LANES)])
      plsc.addupdate_scatter(acc_g, [idx], ch_g[par, pl.ds(o, LANES)])
      plsc.addupdate_scatter(acc_b, [idx], ch_b[par, pl.ds(o, LANES)])
      plsc.addupdate_scatter(acc_c, [idx], ones)

  def fire_p3(i):
    par = i % 2
    off = pix_base + i * CH
    sem = sems[par]
    return [
        pltpu.async_copy(slic_hbm.at[pl.ds(off, CH)], slic_buf.at[par], sem),
        pltpu.async_copy(ndvi_hbm.at[pl.ds(off, CH)], ndvi_buf.at[par], sem),
    ]
  descs3 = [fire_p3(0), None]

  @plsc.parallel_loop(0, NSEG, step=LANES, unroll=2)
  def _(o):
    for a, ref in enumerate((acc_r, acc_g, acc_b, acc_c)):
      tot = ref[pl.ds(o, LANES)]
      for l in range(1, LANES):
        tot = tot + ref[pl.ds(l * NSEG + o, LANES)]
      partial[a, pl.ds(o, LANES)] = tot

  pltpu.sync_copy(partial, shared_acc.at[sub])
  plsc.subcore_barrier()

  gbase = lb * NWPB
  soff = part * (NSEG // NWPB)
  pltpu.sync_copy(
      shared_acc.at[pl.ds(gbase, NWPB), :, pl.ds(soff, NSEG // NWPB)], buf8)

  pltpu.sync_copy(wc_hbm, wc_v.at[pl.ds(0, 9)])
  pltpu.sync_copy(b_hbm, bias_v.at[pl.ds(0, 3)])
  wc_vec = wc_v[pl.ds(0, LANES)]
  bias_vec = bias_v[pl.ds(0, LANES)]
  wc = [wc_vec[k] for k in range(9)]
  bs = [bias_vec[k] for k in range(3)]

  @plsc.parallel_loop(0, NSEG // NWPB, step=LANES, unroll=2)
  def _(o):
    sums = []
    for a in range(4):
      tot = buf8[0, a, pl.ds(o, LANES)]
      for k in range(1, NWPB):
        tot = tot + buf8[k, a, pl.ds(o, LANES)]
      sums.append(tot)
    inv = 1.0 / jnp.maximum(sums[3], 1.0)
    mr = sums[0] * inv
    mg = sums[1] * inv
    mb = sums[2] * inv
    l0 = mr * wc[0] + mg * wc[3] + mb * wc[6] + bs[0]
    l1 = mr * wc[1] + mg * wc[4] + mb * wc[7] + bs[1]
    l2 = mr * wc[2] + mg * wc[5] + mb * wc[8] + bs[2]
    mx = jnp.maximum(l0, jnp.maximum(l1, l2))
    e0 = jnp.exp(l0 - mx)
    e1 = jnp.exp(l1 - mx)
    e2 = jnp.exp(l2 - mx)
    tbl_slice[pl.ds(o, LANES)] = (e1 + 2.0 * e2) / (e0 + e1 + e2) + 1.0

  pltpu.sync_copy(tbl_slice, shared_tbl.at[lb, pl.ds(soff, NSEG // NWPB)])
  plsc.subcore_barrier()
  pltpu.sync_copy(shared_tbl.at[lb], tbl_local)

  out_descs = [None, None]
  for i in range(NCHUNK):
    par = i % 2
    if i + 1 < NCHUNK:
      descs3[1 - par] = fire_p3(i + 1)
    for d in descs3[par]:
      d.wait()
    if out_descs[par] is not None:
      out_descs[par].wait()

    @plsc.parallel_loop(0, CH, step=LANES, unroll=UNROLL)
    def _(o, par=par):
      sl = slic_buf[par, pl.ds(o, LANES)]
      nd = ndvi_buf[par, pl.ds(o, LANES)]
      m = nd > NDVI_THRESH
      seg = jnp.where(m, sl, 0)
      lbl = plsc.load_gather(tbl_local, [seg])
      res = jnp.where(seg > 0, lbl, jnp.where(m, ones, zf))
      out_buf[par, pl.ds(o, LANES)] = res

    off = pix_base + i * CH
    out_descs[par] = pltpu.async_copy(
        out_buf.at[par], out_hbm.at[pl.ds(off, CH)], sem_o)
  out_descs[0].wait()
  out_descs[1].wait()


@jax.jit
def _run(img_flat, ndvi_flat, slic_flat, wc_flat, bias):
  mesh = plsc.VectorSubcoreMesh(core_axis_name="c", subcore_axis_name="s")
  f = pl.kernel(
      _sc_body,
      out_type=jax.ShapeDtypeStruct((NPIX,), jnp.float32),
      mesh=mesh,
      compiler_params=pltpu.CompilerParams(needs_layout_passes=False),
      scratch_types=[
          pltpu.VMEM((LANES * NSEG,), jnp.float32),
          pltpu.VMEM((LANES * NSEG,), jnp.float32),
          pltpu.VMEM((LANES * NSEG,), jnp.float32),
          pltpu.VMEM((LANES * NSEG,), jnp.float32),
          pltpu.VMEM((2, CH), jnp.int32),
          pltpu.VMEM((2, CH), jnp.float32),
          pltpu.VMEM((2, CH), jnp.float32),
          pltpu.VMEM((2, CH), jnp.float32),
          pltpu.VMEM((2, CH), jnp.float32),
          pltpu.VMEM((4, NSEG), jnp.float32),
          pltpu.VMEM((NWPB, 4, NSEG // NWPB), jnp.float32),
          pltpu.VMEM((NSEG // NWPB,), jnp.float32),
          pltpu.VMEM((NSEG,), jnp.float32),
          pltpu.VMEM((2, CH), jnp.float32),
          pltpu.VMEM((LANES,), jnp.float32),
          pltpu.VMEM((LANES,), jnp.float32),
          pltpu.VMEM_SHARED((16, 4, NSEG), jnp.float32),
          pltpu.VMEM_SHARED((2, NSEG), jnp.float32),
          pltpu.SemaphoreType.DMA,
          pltpu.SemaphoreType.DMA,
          pltpu.SemaphoreType.DMA,
      ],
  )
  return f(img_flat, ndvi_flat, slic_flat, wc_flat, bias)


def kernel(image, ndvi, slic, Wc, b):
  out = _run(image.reshape(-1), ndvi.reshape(-1), slic.reshape(-1),
             Wc.reshape(-1), b)
  return out.reshape(B, 1, H, W)


# speedup vs baseline: 151.5539x; 1.0778x over previous
import jax
import jax.numpy as jnp
from jax import lax
from jax.experimental import pallas as pl
from jax.experimental.pallas import tpu as pltpu
from jax.experimental.pallas import tpu_sc as plsc

B = 4
C = 3
H = 512
W = 512
HW = H * W
NPIX = B * HW
NSEG = 1024
NCLS = 3
NDVI_THRESH = 0.5

NWPB = 8
PPW = HW // NWPB
CH = 4096
NCHUNK = PPW // CH
VPC = CH // 16
LANES = 16
UNROLL = 4


def _sc_body(img_hbm, ndvi_hbm, slic_hbm, wc_hbm, b_hbm, out_hbm,
             acc_r, acc_g, acc_b, acc_c,
             slic_0, slic_1, ndvi_0, ndvi_1, ch_r0, ch_r1, ch_g0, ch_g1,
             ch_b0, ch_b1, out_0, out_1,
             partial, buf8, tbl_slice, tbl_local,
             wc_v, bias_v, shared_acc, shared_tbl,
             sem_a, sem_b, sem_o):
  slic_buf = (slic_0, slic_1)
  ndvi_buf = (ndvi_0, ndvi_1)
  ch_r = (ch_r0, ch_r1)
  ch_g = (ch_g0, ch_g1)
  ch_b = (ch_b0, ch_b1)
  out_buf = (out_0, out_1)
  core = lax.axis_index("c")
  sub = lax.axis_index("s")
  lb = sub // NWPB
  part = sub % NWPB
  batch = core * 2 + lb
  pix_base = batch * HW + part * PPW
  img_base = (batch * C) * HW + part * PPW

  lane = lax.iota(jnp.int32, LANES)
  lane_base = lane * NSEG
  ones = jnp.full((LANES,), 1.0, dtype=jnp.float32)
  zf = jnp.zeros((LANES,), dtype=jnp.float32)
  sems = (sem_a, sem_b)

  def fire_p1(i):
    par = i % 2
    off = pix_base + i * CH
    coff = img_base + i * CH
    sem = sems[par]
    return [
        pltpu.async_copy(slic_hbm.at[pl.ds(off, CH)], slic_buf[par], sem),
        pltpu.async_copy(ndvi_hbm.at[pl.ds(off, CH)], ndvi_buf[par], sem),
        pltpu.async_copy(img_hbm.at[pl.ds(coff, CH)], ch_r[par], sem),
        pltpu.async_copy(img_hbm.at[pl.ds(coff + HW, CH)], ch_g[par], sem),
        pltpu.async_copy(img_hbm.at[pl.ds(coff + 2 * HW, CH)], ch_b[par],
                         sem),
    ]

  descs = [fire_p1(0), None]

  @plsc.parallel_loop(0, LANES * NSEG, step=4 * LANES, unroll=4)
  def _(i):
    for j in range(4):
      o = i + j * LANES
      acc_r[pl.ds(o, LANES)] = zf
      acc_g[pl.ds(o, LANES)] = zf
      acc_b[pl.ds(o, LANES)] = zf
      acc_c[pl.ds(o, LANES)] = zf

  for i in range(NCHUNK):
    par = i % 2
    if i + 1 < NCHUNK:
      descs[1 - par] = fire_p1(i + 1)
    for d in descs[par]:
      d.wait()

    @plsc.parallel_loop(0, CH, step=LANES, unroll=UNROLL)
    def _(o, par=par):
      sl = slic_buf[par][pl.ds(o, LANES)]
      nd = ndvi_buf[par][pl.ds(o, LANES)]
      m = nd > NDVI_THRESH
      seg = jnp.where(m, sl, 0)
      idx = seg + lane_base
      plsc.addupdate_scatter(acc_r, [idx], ch_r[par][pl.ds(o, LANES)])
      plsc.addupdate_scatter(acc_g, [idx], ch_g[par][pl.ds(o, LANES)])
      plsc.addupdate_scatter(acc_b, [idx], ch_b[par][pl.ds(o, LANES)])
      plsc.addupdate_scatter(acc_c, [idx], ones)

  def fire_p3(i):
    par = i % 2
    off = pix_base + i * CH
    sem = sems[par]
    return [
        pltpu.async_copy(slic_hbm.at[pl.ds(off, CH)], slic_buf[par], sem),
        pltpu.async_copy(ndvi_hbm.at[pl.ds(off, CH)], ndvi_buf[par], sem),
    ]
  descs3 = [fire_p3(0), None]

  @plsc.parallel_loop(0, NSEG, step=LANES, unroll=2)
  def _(o):
    for a, ref in enumerate((acc_r, acc_g, acc_b, acc_c)):
      tot = ref[pl.ds(o, LANES)]
      for l in range(1, LANES):
        tot = tot + ref[pl.ds(l * NSEG + o, LANES)]
      partial[a, pl.ds(o, LANES)] = tot

  pltpu.sync_copy(partial, shared_acc.at[sub])
  plsc.subcore_barrier()

  gbase = lb * NWPB
  soff = part * (NSEG // NWPB)
  pltpu.sync_copy(
      shared_acc.at[pl.ds(gbase, NWPB), :, pl.ds(soff, NSEG // NWPB)], buf8)

  pltpu.sync_copy(wc_hbm, wc_v.at[pl.ds(0, 9)])
  pltpu.sync_copy(b_hbm, bias_v.at[pl.ds(0, 3)])
  wc_vec = wc_v[pl.ds(0, LANES)]
  bias_vec = bias_v[pl.ds(0, LANES)]
  wc = [wc_vec[k] for k in range(9)]
  bs = [bias_vec[k] for k in range(3)]

  @plsc.parallel_loop(0, NSEG // NWPB, step=LANES, unroll=2)
  def _(o):
    sums = []
    for a in range(4):
      tot = buf8[0, a, pl.ds(o, LANES)]
      for k in range(1, NWPB):
        tot = tot + buf8[k, a, pl.ds(o, LANES)]
      sums.append(tot)
    inv = 1.0 / jnp.maximum(sums[3], 1.0)
    mr = sums[0] * inv
    mg = sums[1] * inv
    mb = sums[2] * inv
    l0 = mr * wc[0] + mg * wc[3] + mb * wc[6] + bs[0]
    l1 = mr * wc[1] + mg * wc[4] + mb * wc[7] + bs[1]
    l2 = mr * wc[2] + mg * wc[5] + mb * wc[8] + bs[2]
    mx = jnp.maximum(l0, jnp.maximum(l1, l2))
    e0 = jnp.exp(l0 - mx)
    e1 = jnp.exp(l1 - mx)
    e2 = jnp.exp(l2 - mx)
    tbl_slice[pl.ds(o, LANES)] = (e1 + 2.0 * e2) / (e0 + e1 + e2) + 1.0

  pltpu.sync_copy(tbl_slice, shared_tbl.at[lb, pl.ds(soff, NSEG // NWPB)])
  plsc.subcore_barrier()
  pltpu.sync_copy(shared_tbl.at[lb], tbl_local)

  out_descs = [None, None]
  for i in range(NCHUNK):
    par = i % 2
    if i + 1 < NCHUNK:
      descs3[1 - par] = fire_p3(i + 1)
    for d in descs3[par]:
      d.wait()
    if out_descs[par] is not None:
      out_descs[par].wait()

    @plsc.parallel_loop(0, CH, step=LANES, unroll=UNROLL)
    def _(o, par=par):
      sl = slic_buf[par][pl.ds(o, LANES)]
      nd = ndvi_buf[par][pl.ds(o, LANES)]
      m = nd > NDVI_THRESH
      seg = jnp.where(m, sl, 0)
      lbl = plsc.load_gather(tbl_local, [seg])
      res = jnp.where(seg > 0, lbl, jnp.where(m, ones, zf))
      out_buf[par][pl.ds(o, LANES)] = res

    off = pix_base + i * CH
    out_descs[par] = pltpu.async_copy(
        out_buf[par], out_hbm.at[pl.ds(off, CH)], sem_o)
  out_descs[0].wait()
  out_descs[1].wait()


@jax.jit
def _run(img_flat, ndvi_flat, slic_flat, wc_flat, bias):
  mesh = plsc.VectorSubcoreMesh(core_axis_name="c", subcore_axis_name="s")
  f = pl.kernel(
      _sc_body,
      out_type=jax.ShapeDtypeStruct((NPIX,), jnp.float32),
      mesh=mesh,
      compiler_params=pltpu.CompilerParams(needs_layout_passes=False),
      scratch_types=[
          pltpu.VMEM((LANES * NSEG,), jnp.float32),
          pltpu.VMEM((LANES * NSEG,), jnp.float32),
          pltpu.VMEM((LANES * NSEG,), jnp.float32),
          pltpu.VMEM((LANES * NSEG,), jnp.float32),
          pltpu.VMEM((CH,), jnp.int32),
          pltpu.VMEM((CH,), jnp.int32),
          pltpu.VMEM((CH,), jnp.float32),
          pltpu.VMEM((CH,), jnp.float32),
          pltpu.VMEM((CH,), jnp.float32),
          pltpu.VMEM((CH,), jnp.float32),
          pltpu.VMEM((CH,), jnp.float32),
          pltpu.VMEM((CH,), jnp.float32),
          pltpu.VMEM((CH,), jnp.float32),
          pltpu.VMEM((CH,), jnp.float32),
          pltpu.VMEM((CH,), jnp.float32),
          pltpu.VMEM((CH,), jnp.float32),
          pltpu.VMEM((4, NSEG), jnp.float32),
          pltpu.VMEM((NWPB, 4, NSEG // NWPB), jnp.float32),
          pltpu.VMEM((NSEG // NWPB,), jnp.float32),
          pltpu.VMEM((NSEG,), jnp.float32),
          pltpu.VMEM((LANES,), jnp.float32),
          pltpu.VMEM((LANES,), jnp.float32),
          pltpu.VMEM_SHARED((16, 4, NSEG), jnp.float32),
          pltpu.VMEM_SHARED((2, NSEG), jnp.float32),
          pltpu.SemaphoreType.DMA,
          pltpu.SemaphoreType.DMA,
          pltpu.SemaphoreType.DMA,
      ],
  )
  return f(img_flat, ndvi_flat, slic_flat, wc_flat, bias)


def kernel(image, ndvi, slic, Wc, b):
  out = _run(image.reshape(-1), ndvi.reshape(-1), slic.reshape(-1),
             Wc.reshape(-1), b)
  return out.reshape(B, 1, H, W)
